# Initial kernel scaffold; baseline (speedup 1.0000x reference)
import jax
import jax.numpy as jnp
from jax.experimental import pallas as pl


def kernel(node_feat, edge_attr, W_node, W_edge, att_src, att_dst, att_edge, ln_gamma, ln_beta, edge_index):
    raise NotImplementedError("write your pallas kernel here")


# speedup vs baseline: 17.2080x; 17.2080x over previous
import functools

import jax
import jax.numpy as jnp
from jax import lax
from jax.experimental import pallas as pl
from jax.experimental.pallas import tpu as pltpu
from jax.experimental.pallas import tpu_sc as plsc

N_NODES = 10000
N_EDGES = 320000
NODE_DIM = 128
EDGE_DIM = 16
HIDDEN = 128
HEADS = 4
HEAD_DIM = HIDDEN // HEADS

NC = 2
NS = 16
NW = NC * NS
L = 16

EPW1 = N_EDGES // NS
BLK1 = 160
NB1 = EPW1 // BLK1
CH1 = 80
EPW2 = N_EDGES // NS
BLK2 = 80
NB2 = EPW2 // BLK2
SPMH_R = 5120
RPT = SPMH_R // NS
ZCH = 16
SPME_R = 2560
RPT2 = SPME_R // NS

_f32 = jnp.float32
_i32 = jnp.int32


def _node_proj_body(nf_ref, wn_ref, asd_ref, h_ref, a_ref, m_ref):
    h = lax.dot_general(nf_ref[...], wn_ref[...],
                        (((1,), (1,)), ((), ())),
                        preferred_element_type=_f32)
    h_ref[...] = h
    a = lax.dot_general(h, asd_ref[...], (((1,), (0,)), ((), ())),
                        preferred_element_type=_f32)
    a_ref[...] = a
    m_ref[...] = jnp.broadcast_to(jnp.max(a, axis=0, keepdims=True), (8, 8))


def _node_proj(node_feat, w_node, a_sd):
    return pl.pallas_call(
        _node_proj_body,
        out_shape=(
            jax.ShapeDtypeStruct((N_NODES, HIDDEN), _f32),
            jax.ShapeDtypeStruct((N_NODES, 8), _f32),
            jax.ShapeDtypeStruct((8, 8), _f32),
        ),
    )(node_feat, w_node, a_sd)


def _edge_coef_body(eat_ref, we_ref, ae_ref, o_ref, m_ref):
    c = lax.dot_general(we_ref[...], ae_ref[...], (((0,), (0,)), ((), ())),
                        preferred_element_type=_f32)
    o = lax.dot_general(c, eat_ref[...], (((0,), (0,)), ((), ())),
                        preferred_element_type=_f32)
    o_ref[...] = o
    m_ref[...] = jnp.broadcast_to(jnp.max(o, axis=1, keepdims=True), (HEADS, 8))


def _edge_coef(edge_attr_t, w_edge, a_edge):
    return pl.pallas_call(
        _edge_coef_body,
        out_shape=(
            jax.ShapeDtypeStruct((HEADS, N_EDGES), _f32),
            jax.ShapeDtypeStruct((HEADS, 8), _f32),
        ),
    )(edge_attr_t, w_edge, a_edge)


def _pass1_body(src_hbm, dst_hbm, an_hbm, ae_hbm, mb_hbm, eat_hbm,
                p_hbm, dpart_hbm, u_hbm,
                an_v, den_v, src_v, dst_v, ae_v, p_v, mb_v, eat_v,
                q_v, didx4_v, msge_v, spme, sem):
    cid = lax.axis_index("c")
    sid = lax.axis_index("s")
    wid = sid * NC + cid
    base = sid * EPW1
    h0 = 2 * cid * N_EDGES

    pltpu.sync_copy(an_hbm.at[pl.ds(cid * N_NODES * 4, N_NODES * 4)], an_v)
    pltpu.sync_copy(mb_hbm.at[pl.ds(cid * 2 * L, 2 * L)], mb_v)
    mb_b = [mb_v[pl.ds(j * L, L)] for j in range(2)]

    def _zero(i, carry):
        den_v[pl.ds(i * L, L)] = jnp.zeros((L,), _f32)
        return carry
    lax.fori_loop(0, N_NODES * 2 // L, _zero, 0)

    def _zmsg(e, carry):
        for c in range(HIDDEN // L):
            msge_v[e, pl.ds(c * L, L)] = jnp.zeros((L,), _f32)
        return carry
    lax.fori_loop(0, BLK1, _zmsg, 0)

    def _zspm(j, carry):
        pltpu.sync_copy(msge_v.at[pl.ds(0, ZCH)],
                        spme.at[pl.ds(sid * RPT2 + j * ZCH, ZCH)])
        return carry
    lax.fori_loop(0, RPT2 // ZCH, _zspm, 0)
    plsc.subcore_barrier()

    def _block(i, carry):
        off = base + i * BLK1
        pltpu.sync_copy(src_hbm.at[pl.ds(off, BLK1)], src_v)
        pltpu.sync_copy(dst_hbm.at[pl.ds(off, BLK1)], dst_v)
        pltpu.sync_copy(eat_hbm.at[pl.ds(off * EDGE_DIM, BLK1 * EDGE_DIM)],
                        eat_v)
        for j in range(2):
            pltpu.sync_copy(ae_hbm.at[pl.ds(h0 + j * N_EDGES + off, BLK1)],
                            ae_v.at[pl.ds(j * BLK1, BLK1)])

        def _grp(g, c2):
            s16 = src_v[pl.ds(g * L, L)]
            d16 = dst_v[pl.ds(g * L, L)]
            q_v[pl.ds(g * L, L)] = d16 & 3
            d4 = lax.shift_right_logical(d16, 2)
            didx4_v[g // (CH1 // L), pl.ds((g % (CH1 // L)) * L, L)] = d4
            for j in range(2):
                asrc = plsc.load_gather(an_v, [s16 * 4 + j])
                adst = plsc.load_gather(an_v, [d16 * 4 + (2 + j)])
                ae16 = ae_v[pl.ds(j * BLK1 + g * L, L)]
                lg = asrc + adst + ae16
                lg = jnp.where(lg >= 0.0, lg, lg * jnp.float32(0.2))
                pexp = jnp.exp(lg - mb_b[j])
                p_v[pl.ds(j * BLK1 + g * L, L)] = pexp
                plsc.addupdate_scatter(den_v, [d16 * 2 + j], pexp)
            return c2
        lax.fori_loop(0, BLK1 // L, _grp, 0)

        def _edge(e, c2):
            eidx = jnp.broadcast_to(e, (L,))
            pb = [plsc.load_gather(p_v, [eidx + (j * BLK1)]) for j in range(2)]
            qb = plsc.load_gather(q_v, [eidx])
            eav = eat_v[pl.ds(e * EDGE_DIM, EDGE_DIM)]
            for j in range(2):
                v = eav * pb[j]
                for q in range(4):
                    mq = jnp.where(qb == q, jnp.float32(1.0), jnp.float32(0.0))
                    msge_v[e, pl.ds(q * 2 * EDGE_DIM + j * EDGE_DIM,
                                    EDGE_DIM)] = v * mq
            return c2
        lax.fori_loop(0, BLK1, _edge, 0)

        for c in range(BLK1 // CH1):
            pltpu.sync_copy(msge_v.at[pl.ds(c * CH1, CH1)],
                            spme.at[didx4_v.at[c]], add=True)
        for j in range(2):
            pltpu.sync_copy(p_v.at[pl.ds(j * BLK1, BLK1)],
                            p_hbm.at[pl.ds(h0 + j * N_EDGES + off, BLK1)])
        return carry
    lax.fori_loop(0, NB1, _block, 0)

    pltpu.sync_copy(den_v, dpart_hbm.at[pl.ds(wid * N_NODES * 2,
                                              N_NODES * 2)])
    plsc.subcore_barrier()

    def _flush(j, carry):
        pltpu.sync_copy(spme.at[pl.ds(sid * RPT2 + j * ZCH, ZCH)],
                        u_hbm.at[pl.ds(cid * SPME_R + sid * RPT2 + j * ZCH,
                                       ZCH)])
        return carry
    lax.fori_loop(0, RPT2 // ZCH, _flush, 0)


def _pass1(src, dst, an_sp, ae_t, mb, ea_flat):
    f = functools.partial(
        pl.kernel,
        out_type=(
            jax.ShapeDtypeStruct((HEADS * N_EDGES,), _f32),
            jax.ShapeDtypeStruct((NW * N_NODES * 2,), _f32),
            jax.ShapeDtypeStruct((NC * SPME_R, HIDDEN), _f32),
        ),
        mesh=plsc.VectorSubcoreMesh(core_axis_name="c", subcore_axis_name="s",
                                    num_cores=NC, num_subcores=NS),
        compiler_params=pltpu.CompilerParams(needs_layout_passes=False),
        scratch_types=[
            pltpu.VMEM((N_NODES * 4,), _f32),
            pltpu.VMEM((N_NODES * 2,), _f32),
            pltpu.VMEM((BLK1,), _i32),
            pltpu.VMEM((BLK1,), _i32),
            pltpu.VMEM((BLK1 * 2,), _f32),
            pltpu.VMEM((BLK1 * 2,), _f32),
            pltpu.VMEM((2 * L,), _f32),
            pltpu.VMEM((BLK1 * EDGE_DIM,), _f32),
            pltpu.VMEM((BLK1,), _i32),
            pltpu.VMEM((BLK1 // CH1, CH1), _i32),
            pltpu.VMEM((BLK1, HIDDEN), _f32),
            pltpu.VMEM_SHARED((SPME_R, HIDDEN), _f32),
            pltpu.SemaphoreType.DMA,
        ],
    )
    return f(_pass1_body)(src, dst, an_sp, ae_t, mb, ea_flat)


def _dinv_body(dp_ref, o_ref):
    s = jnp.sum(dp_ref[...], axis=0, keepdims=True)
    o_ref[...] = 1.0 / s


def _dinv(dparts):
    return pl.pallas_call(
        _dinv_body,
        out_shape=jax.ShapeDtypeStruct((1, NC * N_NODES * 2), _f32),
    )(dparts)


def _pass2_body(src_hbm, dst_hbm, p_hbm, din_hbm, h_hbm,
                out1_hbm,
                din_v, sidx_v, didx_v, didx2_v, par_v, p_v, al_v,
                hrow_v, msgh_v, spmh, sem):
    cid = lax.axis_index("c")
    sid = lax.axis_index("s")
    base = sid * EPW2
    h0 = 2 * cid * N_EDGES

    pltpu.sync_copy(din_hbm.at[pl.ds(cid * N_NODES * 2, N_NODES * 2)], din_v)

    row1 = sid * RPT

    def _zmsg(e, carry):
        for c in range(HIDDEN // L):
            msgh_v[e, pl.ds(c * L, L)] = jnp.zeros((L,), _f32)
        return carry
    lax.fori_loop(0, BLK2, _zmsg, 0)

    def _zspm1(j, carry):
        pltpu.sync_copy(msgh_v.at[pl.ds(0, ZCH)],
                        spmh.at[pl.ds(row1 + j * ZCH, ZCH)])
        return carry
    lax.fori_loop(0, RPT // ZCH, _zspm1, 0)
    plsc.subcore_barrier()

    def _block(i, carry):
        off = base + i * BLK2
        pltpu.sync_copy(src_hbm.at[pl.ds(off, BLK2)], sidx_v)
        pltpu.sync_copy(dst_hbm.at[pl.ds(off, BLK2)], didx_v)
        for j in range(2):
            pltpu.sync_copy(p_hbm.at[pl.ds(h0 + j * N_EDGES + off, BLK2)],
                            p_v.at[pl.ds(j * BLK2, BLK2)])

        def _alpha(g, c2):
            d16 = didx_v[pl.ds(g * L, L)]
            didx2_v[pl.ds(g * L, L)] = lax.shift_right_logical(d16, 1)
            par_v[pl.ds(g * L, L)] = (d16 & 1).astype(_f32)
            for j in range(2):
                pe = p_v[pl.ds(j * BLK2 + g * L, L)]
                dv = plsc.load_gather(din_v, [d16 * 2 + j])
                al_v[pl.ds(j * BLK2 + g * L, L)] = pe * dv
            return c2
        lax.fori_loop(0, BLK2 // L, _alpha, 0)

        pltpu.async_copy(h_hbm.at[sidx_v], hrow_v, sem).wait()

        def _edge(e, c2):
            eidx = jnp.broadcast_to(e, (L,))
            ab = [plsc.load_gather(al_v, [eidx + (j * BLK2)]) for j in range(2)]
            par = plsc.load_gather(par_v, [eidx])
            npar = 1.0 - par
            for k in range(64 // L):
                hv = hrow_v[e, pl.ds(cid * 64 + k * L, L)]
                v = hv * ab[k // 2]
                msgh_v[e, pl.ds(k * L, L)] = v * npar
                msgh_v[e, pl.ds(64 + k * L, L)] = v * par
            return c2
        lax.fori_loop(0, BLK2, _edge, 0)

        pltpu.sync_copy(msgh_v, spmh.at[didx2_v], add=True)
        return carry
    lax.fori_loop(0, NB2, _block, 0)
    plsc.subcore_barrier()

    def _flush1(j, carry):
        pltpu.sync_copy(spmh.at[pl.ds(row1 + j * ZCH, ZCH)],
                        out1_hbm.at[pl.ds(cid * SPMH_R + row1 + j * ZCH, ZCH)])
        return carry
    lax.fori_loop(0, RPT // ZCH, _flush1, 0)


def _pass2(src, dst, p, dinv, h_sp):
    f = functools.partial(
        pl.kernel,
        out_type=jax.ShapeDtypeStruct((NC * SPMH_R, HIDDEN), _f32),
        mesh=plsc.VectorSubcoreMesh(core_axis_name="c", subcore_axis_name="s",
                                    num_cores=NC, num_subcores=NS),
        compiler_params=pltpu.CompilerParams(needs_layout_passes=False),
        scratch_types=[
            pltpu.VMEM((N_NODES * 2,), _f32),
            pltpu.VMEM((BLK2,), _i32),
            pltpu.VMEM((BLK2,), _i32),
            pltpu.VMEM((BLK2,), _i32),
            pltpu.VMEM((BLK2,), _f32),
            pltpu.VMEM((BLK2 * 2,), _f32),
            pltpu.VMEM((BLK2 * 2,), _f32),
            pltpu.VMEM((BLK2, HIDDEN), _f32),
            pltpu.VMEM((BLK2, HIDDEN), _f32),
            pltpu.VMEM_SHARED((SPMH_R, HIDDEN), _f32),
            pltpu.SemaphoreType.DMA,
        ],
    )
    return f(_pass2_body)(src, dst, p, dinv, h_sp)


def _final_body(acc1_ref, u_ref, din_ref, nf_ref, we_ref, g_ref, b_ref, o_ref):
    a1 = acc1_ref[...]
    out1 = jnp.concatenate([a1[0], a1[1]], axis=1)
    u = u_ref[...]
    din = din_ref[...]
    o2 = []
    for h in range(HEADS):
        c, j = divmod(h, 2)
        uh = u[c][:, j * EDGE_DIM:(j + 1) * EDGE_DIM]
        bh = uh * din[c][:, j][:, None]
        wh = we_ref[pl.ds(h * HEAD_DIM, HEAD_DIM), :]
        o2.append(lax.dot_general(bh, wh, (((1,), (1,)), ((), ())),
                                  preferred_element_type=_f32))
    out2 = jnp.concatenate(o2, axis=1)
    pre = out1 + out2 + nf_ref[...]
    mu = jnp.mean(pre, axis=1, keepdims=True)
    cen = pre - mu
    var = jnp.mean(cen * cen, axis=1, keepdims=True)
    normed = cen * lax.rsqrt(var + 1e-5) * g_ref[...] + b_ref[...]
    o_ref[...] = jnp.where(normed > 0, normed, jnp.exp(jnp.minimum(normed, 0.0)) - 1.0)


_BR = 2000


def _final(acc1, u2, din3, node_feat, w_edge, gamma, beta):
    return pl.pallas_call(
        _final_body,
        grid=(N_NODES // _BR,),
        in_specs=[
            pl.BlockSpec((NC, _BR, 64), lambda i: (0, i, 0)),
            pl.BlockSpec((NC, _BR, 2 * EDGE_DIM), lambda i: (0, i, 0)),
            pl.BlockSpec((NC, _BR, 2), lambda i: (0, i, 0)),
            pl.BlockSpec((_BR, HIDDEN), lambda i: (i, 0)),
            pl.BlockSpec((HIDDEN, EDGE_DIM), lambda i: (0, 0)),
            pl.BlockSpec((1, HIDDEN), lambda i: (0, 0)),
            pl.BlockSpec((1, HIDDEN), lambda i: (0, 0)),
        ],
        out_specs=pl.BlockSpec((_BR, HIDDEN), lambda i: (i, 0)),
        out_shape=jax.ShapeDtypeStruct((N_NODES, HIDDEN), _f32),
    )(acc1, u2, din3, node_feat, w_edge, gamma, beta)


def kernel(node_feat, edge_attr, W_node, W_edge, att_src, att_dst, att_edge,
           ln_gamma, ln_beta, edge_index):
    src = edge_index[0].astype(_i32)
    dst = edge_index[1].astype(_i32)

    eye = jnp.eye(HEADS, dtype=_f32)
    a_src_m = (eye[:, None, :] * att_src[:, :, None]).reshape(HIDDEN, HEADS)
    a_dst_m = (eye[:, None, :] * att_dst[:, :, None]).reshape(HIDDEN, HEADS)
    a_edge_m = (eye[:, None, :] * att_edge[:, :, None]).reshape(HIDDEN, HEADS)
    a_sd = jnp.concatenate([a_src_m, a_dst_m], axis=1)

    h, a_nodes, m_node = _node_proj(node_feat, W_node, a_sd)
    ae_t, m_edge = _edge_coef(edge_attr.T, W_edge, a_edge_m)

    mb = m_node[0, :HEADS] + m_node[0, HEADS:] + m_edge[:, 0]
    mb = jnp.where(mb >= 0.0, mb, mb * 0.2)
    mb16 = jnp.broadcast_to(mb[:, None], (HEADS, L)).reshape(-1)

    asrc, adst = a_nodes[:, :HEADS], a_nodes[:, HEADS:]
    an_sp = jnp.concatenate(
        [jnp.concatenate([asrc[:, 2 * c:2 * c + 2], adst[:, 2 * c:2 * c + 2]],
                         axis=1).reshape(-1) for c in range(NC)])

    p, dparts, u = _pass1(src, dst, an_sp, ae_t.reshape(-1), mb16,
                          edge_attr.reshape(-1))
    dinv = _dinv(dparts.reshape(NS, NC * N_NODES * 2)).reshape(-1)
    acc1 = _pass2(src, dst, p, dinv, h)
    a1 = acc1.reshape(NC, SPMH_R * 2, 64)[:, :N_NODES, :]
    u2 = u.reshape(NC, SPME_R * 4, 2 * EDGE_DIM)[:, :N_NODES, :]
    return _final(a1, u2,
                  dinv.reshape(NC, N_NODES, 2), node_feat, W_edge,
                  ln_gamma.reshape(1, HIDDEN), ln_beta.reshape(1, HIDDEN))


# speedup vs baseline: 17.2887x; 1.0047x over previous
import functools

import jax
import jax.numpy as jnp
from jax import lax
from jax.experimental import pallas as pl
from jax.experimental.pallas import tpu as pltpu
from jax.experimental.pallas import tpu_sc as plsc

N_NODES = 10000
N_EDGES = 320000
NODE_DIM = 128
EDGE_DIM = 16
HIDDEN = 128
HEADS = 4
HEAD_DIM = HIDDEN // HEADS

NC = 2
NS = 16
NW = NC * NS
L = 16

EPW1 = N_EDGES // NS
BLK1 = 160
NB1 = EPW1 // BLK1
CH1 = 80
EPW2 = N_EDGES // NS
BLK2 = 80
NB2 = EPW2 // BLK2
SPMH_R = 5120
RPT = SPMH_R // NS
ZCH = 16
SPME_R = 2560
RPT2 = SPME_R // NS

_f32 = jnp.float32
_i32 = jnp.int32


def _node_proj_body(nf_ref, wn_ref, asd_ref, h_ref, a_ref, m_ref):
    h = lax.dot_general(nf_ref[...], wn_ref[...],
                        (((1,), (1,)), ((), ())),
                        preferred_element_type=_f32)
    h_ref[...] = h
    a = lax.dot_general(h, asd_ref[...], (((1,), (0,)), ((), ())),
                        preferred_element_type=_f32)
    a_ref[...] = a
    m_ref[...] = jnp.broadcast_to(jnp.max(a, axis=0, keepdims=True), (8, 8))


def _node_proj(node_feat, w_node, a_sd):
    return pl.pallas_call(
        _node_proj_body,
        out_shape=(
            jax.ShapeDtypeStruct((N_NODES, HIDDEN), _f32),
            jax.ShapeDtypeStruct((N_NODES, 8), _f32),
            jax.ShapeDtypeStruct((8, 8), _f32),
        ),
    )(node_feat, w_node, a_sd)


def _edge_coef_body(eat_ref, we_ref, ae_ref, o_ref, m_ref):
    c = lax.dot_general(we_ref[...], ae_ref[...], (((0,), (0,)), ((), ())),
                        preferred_element_type=_f32)
    o = lax.dot_general(c, eat_ref[...], (((0,), (0,)), ((), ())),
                        preferred_element_type=_f32)
    o_ref[...] = o
    m_ref[...] = jnp.broadcast_to(jnp.max(o, axis=1, keepdims=True), (HEADS, 8))


def _edge_coef(edge_attr_t, w_edge, a_edge):
    return pl.pallas_call(
        _edge_coef_body,
        out_shape=(
            jax.ShapeDtypeStruct((HEADS, N_EDGES), _f32),
            jax.ShapeDtypeStruct((HEADS, 8), _f32),
        ),
    )(edge_attr_t, w_edge, a_edge)


def _pass1_body(src_hbm, dst_hbm, an_hbm, ae_hbm, mb_hbm,
                p_hbm, dpart_hbm,
                an_v, den_v, src_v, dst_v, ae_v, p_v, mb_v, sem):
    cid = lax.axis_index("c")
    sid = lax.axis_index("s")
    wid = sid * NC + cid
    base = sid * EPW1
    h0 = 2 * cid * N_EDGES

    pltpu.sync_copy(an_hbm.at[pl.ds(cid * N_NODES * 4, N_NODES * 4)], an_v)
    pltpu.sync_copy(mb_hbm.at[pl.ds(cid * 2 * L, 2 * L)], mb_v)
    mb_b = [mb_v[pl.ds(j * L, L)] for j in range(2)]

    def _zero(i, carry):
        den_v[pl.ds(i * L, L)] = jnp.zeros((L,), _f32)
        return carry
    lax.fori_loop(0, N_NODES * 2 // L, _zero, 0)

    def _block(i, carry):
        off = base + i * BLK1
        pltpu.sync_copy(src_hbm.at[pl.ds(off, BLK1)], src_v)
        pltpu.sync_copy(dst_hbm.at[pl.ds(off, BLK1)], dst_v)
        for j in range(2):
            pltpu.sync_copy(ae_hbm.at[pl.ds(h0 + j * N_EDGES + off, BLK1)],
                            ae_v.at[pl.ds(j * BLK1, BLK1)])

        def _grp(g, c2):
            s16 = src_v[pl.ds(g * L, L)]
            d16 = dst_v[pl.ds(g * L, L)]
            for j in range(2):
                asrc = plsc.load_gather(an_v, [s16 * 4 + j])
                adst = plsc.load_gather(an_v, [d16 * 4 + (2 + j)])
                ae16 = ae_v[pl.ds(j * BLK1 + g * L, L)]
                lg = asrc + adst + ae16
                lg = jnp.where(lg >= 0.0, lg, lg * jnp.float32(0.2))
                pexp = jnp.exp(lg - mb_b[j])
                p_v[pl.ds(j * BLK1 + g * L, L)] = pexp
                plsc.addupdate_scatter(den_v, [d16 * 2 + j], pexp)
            return c2
        lax.fori_loop(0, BLK1 // L, _grp, 0)

        for j in range(2):
            pltpu.sync_copy(p_v.at[pl.ds(j * BLK1, BLK1)],
                            p_hbm.at[pl.ds(h0 + j * N_EDGES + off, BLK1)])
        return carry
    lax.fori_loop(0, NB1, _block, 0)

    pltpu.sync_copy(den_v, dpart_hbm.at[pl.ds(wid * N_NODES * 2,
                                              N_NODES * 2)])


def _pass1(src, dst, an_sp, ae_t, mb):
    f = functools.partial(
        pl.kernel,
        out_type=(
            jax.ShapeDtypeStruct((HEADS * N_EDGES,), _f32),
            jax.ShapeDtypeStruct((NW * N_NODES * 2,), _f32),
        ),
        mesh=plsc.VectorSubcoreMesh(core_axis_name="c", subcore_axis_name="s",
                                    num_cores=NC, num_subcores=NS),
        compiler_params=pltpu.CompilerParams(needs_layout_passes=False),
        scratch_types=[
            pltpu.VMEM((N_NODES * 4,), _f32),
            pltpu.VMEM((N_NODES * 2,), _f32),
            pltpu.VMEM((BLK1,), _i32),
            pltpu.VMEM((BLK1,), _i32),
            pltpu.VMEM((BLK1 * 2,), _f32),
            pltpu.VMEM((BLK1 * 2,), _f32),
            pltpu.VMEM((2 * L,), _f32),
            pltpu.SemaphoreType.DMA,
        ],
    )
    return f(_pass1_body)(src, dst, an_sp, ae_t, mb)


def _dinv_body(dp_ref, o_ref):
    s = jnp.sum(dp_ref[...], axis=0, keepdims=True)
    o_ref[...] = 1.0 / s


def _dinv(dparts):
    return pl.pallas_call(
        _dinv_body,
        out_shape=jax.ShapeDtypeStruct((1, NC * N_NODES * 2), _f32),
    )(dparts)


def _pass2_body(src_hbm, dst_hbm, p_hbm, din_hbm, h_hbm, eat_hbm,
                out1_hbm, u_hbm,
                din_v, sidx_v, didx_v, didx2_v, didx4_v, par_v, q_v,
                p_v, al_v, eat_v, hrow_v, msgh_v, msge_v,
                spmh, spme, sem):
    cid = lax.axis_index("c")
    sid = lax.axis_index("s")
    base = sid * EPW2
    h0 = 2 * cid * N_EDGES

    pltpu.sync_copy(din_hbm.at[pl.ds(cid * N_NODES * 2, N_NODES * 2)], din_v)

    row1 = sid * RPT
    row2 = sid * RPT2

    def _zmsg(e, carry):
        for c in range(HIDDEN // L):
            msgh_v[e, pl.ds(c * L, L)] = jnp.zeros((L,), _f32)
        return carry
    lax.fori_loop(0, BLK2, _zmsg, 0)

    def _zspm1(j, carry):
        pltpu.sync_copy(msgh_v.at[pl.ds(0, ZCH)],
                        spmh.at[pl.ds(row1 + j * ZCH, ZCH)])
        return carry
    lax.fori_loop(0, RPT // ZCH, _zspm1, 0)

    def _zspm2(j, carry):
        pltpu.sync_copy(msgh_v.at[pl.ds(0, ZCH)],
                        spme.at[pl.ds(row2 + j * ZCH, ZCH)])
        return carry
    lax.fori_loop(0, RPT2 // ZCH, _zspm2, 0)
    plsc.subcore_barrier()

    def _block(i, carry):
        off = base + i * BLK2
        pltpu.sync_copy(src_hbm.at[pl.ds(off, BLK2)], sidx_v)
        pltpu.sync_copy(dst_hbm.at[pl.ds(off, BLK2)], didx_v)
        pltpu.sync_copy(eat_hbm.at[pl.ds(off * EDGE_DIM, BLK2 * EDGE_DIM)],
                        eat_v)
        for j in range(2):
            pltpu.sync_copy(p_hbm.at[pl.ds(h0 + j * N_EDGES + off, BLK2)],
                            p_v.at[pl.ds(j * BLK2, BLK2)])

        def _alpha(g, c2):
            d16 = didx_v[pl.ds(g * L, L)]
            didx2_v[pl.ds(g * L, L)] = lax.shift_right_logical(d16, 1)
            didx4_v[pl.ds(g * L, L)] = lax.shift_right_logical(d16, 2)
            par_v[pl.ds(g * L, L)] = (d16 & 1).astype(_f32)
            q_v[pl.ds(g * L, L)] = d16 & 3
            for j in range(2):
                pe = p_v[pl.ds(j * BLK2 + g * L, L)]
                dv = plsc.load_gather(din_v, [d16 * 2 + j])
                al_v[pl.ds(j * BLK2 + g * L, L)] = pe * dv
            return c2
        lax.fori_loop(0, BLK2 // L, _alpha, 0)

        pltpu.async_copy(h_hbm.at[sidx_v], hrow_v, sem).wait()

        def _edge(e, c2):
            eidx = jnp.broadcast_to(e, (L,))
            ab = [plsc.load_gather(al_v, [eidx + (j * BLK2)]) for j in range(2)]
            par = plsc.load_gather(par_v, [eidx])
            npar = 1.0 - par
            for k in range(64 // L):
                hv = hrow_v[e, pl.ds(cid * 64 + k * L, L)]
                v = hv * ab[k // 2]
                msgh_v[e, pl.ds(k * L, L)] = v * npar
                msgh_v[e, pl.ds(64 + k * L, L)] = v * par
            pb = [plsc.load_gather(p_v, [eidx + (j * BLK2)]) for j in range(2)]
            qb = plsc.load_gather(q_v, [eidx])
            eav = eat_v[pl.ds(e * EDGE_DIM, EDGE_DIM)]
            mq = [jnp.where(qb == q, jnp.float32(1.0), jnp.float32(0.0))
                  for q in range(4)]
            for j in range(2):
                v = eav * pb[j]
                for q in range(4):
                    msge_v[e, pl.ds(q * 2 * EDGE_DIM + j * EDGE_DIM,
                                    EDGE_DIM)] = v * mq[q]
            return c2
        lax.fori_loop(0, BLK2, _edge, 0)

        pltpu.sync_copy(msgh_v, spmh.at[didx2_v], add=True)
        pltpu.sync_copy(msge_v, spme.at[didx4_v], add=True)
        return carry
    lax.fori_loop(0, NB2, _block, 0)
    plsc.subcore_barrier()

    def _flush1(j, carry):
        pltpu.sync_copy(spmh.at[pl.ds(row1 + j * ZCH, ZCH)],
                        out1_hbm.at[pl.ds(cid * SPMH_R + row1 + j * ZCH, ZCH)])
        return carry
    lax.fori_loop(0, RPT // ZCH, _flush1, 0)

    def _flush2(j, carry):
        pltpu.sync_copy(spme.at[pl.ds(row2 + j * ZCH, ZCH)],
                        u_hbm.at[pl.ds(cid * SPME_R + row2 + j * ZCH, ZCH)])
        return carry
    lax.fori_loop(0, RPT2 // ZCH, _flush2, 0)


def _pass2(src, dst, p, dinv, h_sp, ea_flat):
    f = functools.partial(
        pl.kernel,
        out_type=(
            jax.ShapeDtypeStruct((NC * SPMH_R, HIDDEN), _f32),
            jax.ShapeDtypeStruct((NC * SPME_R, HIDDEN), _f32),
        ),
        mesh=plsc.VectorSubcoreMesh(core_axis_name="c", subcore_axis_name="s",
                                    num_cores=NC, num_subcores=NS),
        compiler_params=pltpu.CompilerParams(needs_layout_passes=False),
        scratch_types=[
            pltpu.VMEM((N_NODES * 2,), _f32),
            pltpu.VMEM((BLK2,), _i32),
            pltpu.VMEM((BLK2,), _i32),
            pltpu.VMEM((BLK2,), _i32),
            pltpu.VMEM((BLK2,), _i32),
            pltpu.VMEM((BLK2,), _f32),
            pltpu.VMEM((BLK2,), _i32),
            pltpu.VMEM((BLK2 * 2,), _f32),
            pltpu.VMEM((BLK2 * 2,), _f32),
            pltpu.VMEM((BLK2 * EDGE_DIM,), _f32),
            pltpu.VMEM((BLK2, HIDDEN), _f32),
            pltpu.VMEM((BLK2, HIDDEN), _f32),
            pltpu.VMEM((BLK2, HIDDEN), _f32),
            pltpu.VMEM_SHARED((SPMH_R, HIDDEN), _f32),
            pltpu.VMEM_SHARED((SPME_R, HIDDEN), _f32),
            pltpu.SemaphoreType.DMA,
        ],
    )
    return f(_pass2_body)(src, dst, p, dinv, h_sp, ea_flat)


def _final_body(acc1_ref, u_ref, din_ref, nf_ref, we_ref, g_ref, b_ref, o_ref):
    a1 = acc1_ref[...]
    out1 = jnp.concatenate([a1[0], a1[1]], axis=1)
    u = u_ref[...]
    din = din_ref[...]
    o2 = []
    for h in range(HEADS):
        c, j = divmod(h, 2)
        uh = u[c][:, j * EDGE_DIM:(j + 1) * EDGE_DIM]
        bh = uh * din[c][:, j][:, None]
        wh = we_ref[pl.ds(h * HEAD_DIM, HEAD_DIM), :]
        o2.append(lax.dot_general(bh, wh, (((1,), (1,)), ((), ())),
                                  preferred_element_type=_f32))
    out2 = jnp.concatenate(o2, axis=1)
    pre = out1 + out2 + nf_ref[...]
    mu = jnp.mean(pre, axis=1, keepdims=True)
    cen = pre - mu
    var = jnp.mean(cen * cen, axis=1, keepdims=True)
    normed = cen * lax.rsqrt(var + 1e-5) * g_ref[...] + b_ref[...]
    o_ref[...] = jnp.where(normed > 0, normed, jnp.exp(jnp.minimum(normed, 0.0)) - 1.0)


_BR = 2000


def _final(acc1, u2, din3, node_feat, w_edge, gamma, beta):
    return pl.pallas_call(
        _final_body,
        grid=(N_NODES // _BR,),
        in_specs=[
            pl.BlockSpec((NC, _BR, 64), lambda i: (0, i, 0)),
            pl.BlockSpec((NC, _BR, 2 * EDGE_DIM), lambda i: (0, i, 0)),
            pl.BlockSpec((NC, _BR, 2), lambda i: (0, i, 0)),
            pl.BlockSpec((_BR, HIDDEN), lambda i: (i, 0)),
            pl.BlockSpec((HIDDEN, EDGE_DIM), lambda i: (0, 0)),
            pl.BlockSpec((1, HIDDEN), lambda i: (0, 0)),
            pl.BlockSpec((1, HIDDEN), lambda i: (0, 0)),
        ],
        out_specs=pl.BlockSpec((_BR, HIDDEN), lambda i: (i, 0)),
        out_shape=jax.ShapeDtypeStruct((N_NODES, HIDDEN), _f32),
    )(acc1, u2, din3, node_feat, w_edge, gamma, beta)


def kernel(node_feat, edge_attr, W_node, W_edge, att_src, att_dst, att_edge,
           ln_gamma, ln_beta, edge_index):
    src = edge_index[0].astype(_i32)
    dst = edge_index[1].astype(_i32)

    eye = jnp.eye(HEADS, dtype=_f32)
    a_src_m = (eye[:, None, :] * att_src[:, :, None]).reshape(HIDDEN, HEADS)
    a_dst_m = (eye[:, None, :] * att_dst[:, :, None]).reshape(HIDDEN, HEADS)
    a_edge_m = (eye[:, None, :] * att_edge[:, :, None]).reshape(HIDDEN, HEADS)
    a_sd = jnp.concatenate([a_src_m, a_dst_m], axis=1)

    h, a_nodes, m_node = _node_proj(node_feat, W_node, a_sd)
    ae_t, m_edge = _edge_coef(edge_attr.T, W_edge, a_edge_m)

    mb = m_node[0, :HEADS] + m_node[0, HEADS:] + m_edge[:, 0]
    mb = jnp.where(mb >= 0.0, mb, mb * 0.2)
    mb16 = jnp.broadcast_to(mb[:, None], (HEADS, L)).reshape(-1)

    asrc, adst = a_nodes[:, :HEADS], a_nodes[:, HEADS:]
    an_sp = jnp.concatenate(
        [jnp.concatenate([asrc[:, 2 * c:2 * c + 2], adst[:, 2 * c:2 * c + 2]],
                         axis=1).reshape(-1) for c in range(NC)])

    p, dparts = _pass1(src, dst, an_sp, ae_t.reshape(-1), mb16)
    dinv = _dinv(dparts.reshape(NS, NC * N_NODES * 2)).reshape(-1)
    acc1, u = _pass2(src, dst, p, dinv, h, edge_attr.reshape(-1))
    a1 = acc1.reshape(NC, SPMH_R * 2, 64)[:, :N_NODES, :]
    u2 = u.reshape(NC, SPME_R * 4, 2 * EDGE_DIM)[:, :N_NODES, :]
    return _final(a1, u2,
                  dinv.reshape(NC, N_NODES, 2), node_feat, W_edge,
                  ln_gamma.reshape(1, HIDDEN), ln_beta.reshape(1, HIDDEN))


# speedup vs baseline: 23.1487x; 1.3389x over previous
import functools

import jax
import jax.numpy as jnp
from jax import lax
from jax.experimental import pallas as pl
from jax.experimental.pallas import tpu as pltpu
from jax.experimental.pallas import tpu_sc as plsc

N_NODES = 10000
N_EDGES = 320000
NODE_DIM = 128
EDGE_DIM = 16
HIDDEN = 128
HEADS = 4
HEAD_DIM = HIDDEN // HEADS

NC = 2
NS = 16
NW = NC * NS
L = 16

EPW1 = N_EDGES // NS
BLK1 = 160
NB1 = EPW1 // BLK1
CH1 = 80
EPW2 = N_EDGES // NS
BLK2 = 160
NB2 = EPW2 // BLK2
SPMH_R = 5120
RPT = SPMH_R // NS
ZCH = 16
SPME_R = 2560
RPT2 = SPME_R // NS

_f32 = jnp.float32
_i32 = jnp.int32


def _node_proj_body(nf_ref, wn_ref, asd_ref, h_ref, a_ref, m_ref):
    h = lax.dot_general(nf_ref[...], wn_ref[...],
                        (((1,), (1,)), ((), ())),
                        preferred_element_type=_f32)
    h_ref[...] = h
    a = lax.dot_general(h, asd_ref[...], (((1,), (0,)), ((), ())),
                        preferred_element_type=_f32)
    a_ref[...] = a
    m_ref[...] = jnp.broadcast_to(jnp.max(a, axis=0, keepdims=True), (8, 8))


def _node_proj(node_feat, w_node, a_sd):
    return pl.pallas_call(
        _node_proj_body,
        out_shape=(
            jax.ShapeDtypeStruct((N_NODES, HIDDEN), _f32),
            jax.ShapeDtypeStruct((N_NODES, 8), _f32),
            jax.ShapeDtypeStruct((8, 8), _f32),
        ),
    )(node_feat, w_node, a_sd)


def _edge_coef_body(eat_ref, we_ref, ae_ref, o_ref, m_ref):
    c = lax.dot_general(we_ref[...], ae_ref[...], (((0,), (0,)), ((), ())),
                        preferred_element_type=_f32)
    o = lax.dot_general(c, eat_ref[...], (((0,), (0,)), ((), ())),
                        preferred_element_type=_f32)
    o_ref[...] = o
    m_ref[...] = jnp.broadcast_to(jnp.max(o, axis=1, keepdims=True), (HEADS, 8))


def _edge_coef(edge_attr_t, w_edge, a_edge):
    return pl.pallas_call(
        _edge_coef_body,
        out_shape=(
            jax.ShapeDtypeStruct((HEADS, N_EDGES), _f32),
            jax.ShapeDtypeStruct((HEADS, 8), _f32),
        ),
    )(edge_attr_t, w_edge, a_edge)


def _pass1_body(src_hbm, dst_hbm, an_hbm, ae_hbm, mb_hbm,
                p_hbm, dpart_hbm,
                an_v, den_v, src_v, dst_v, ae_v, p_v, mb_v, sem):
    cid = lax.axis_index("c")
    sid = lax.axis_index("s")
    wid = sid * NC + cid
    base = sid * EPW1
    h0 = 2 * cid * N_EDGES

    pltpu.sync_copy(an_hbm.at[pl.ds(cid * N_NODES * 4, N_NODES * 4)], an_v)
    pltpu.sync_copy(mb_hbm.at[pl.ds(cid * 2 * L, 2 * L)], mb_v)
    mb_b = [mb_v[pl.ds(j * L, L)] for j in range(2)]

    def _zero(i, carry):
        den_v[pl.ds(i * L, L)] = jnp.zeros((L,), _f32)
        return carry
    lax.fori_loop(0, N_NODES * 2 // L, _zero, 0)

    def _block(i, carry):
        off = base + i * BLK1
        pltpu.sync_copy(src_hbm.at[pl.ds(off, BLK1)], src_v)
        pltpu.sync_copy(dst_hbm.at[pl.ds(off, BLK1)], dst_v)
        for j in range(2):
            pltpu.sync_copy(ae_hbm.at[pl.ds(h0 + j * N_EDGES + off, BLK1)],
                            ae_v.at[pl.ds(j * BLK1, BLK1)])

        def _grp(g, c2):
            s16 = src_v[pl.ds(g * L, L)]
            d16 = dst_v[pl.ds(g * L, L)]
            for j in range(2):
                asrc = plsc.load_gather(an_v, [s16 * 4 + j])
                adst = plsc.load_gather(an_v, [d16 * 4 + (2 + j)])
                ae16 = ae_v[pl.ds(j * BLK1 + g * L, L)]
                lg = asrc + adst + ae16
                lg = jnp.where(lg >= 0.0, lg, lg * jnp.float32(0.2))
                pexp = jnp.exp(lg - mb_b[j])
                p_v[pl.ds(j * BLK1 + g * L, L)] = pexp
                plsc.addupdate_scatter(den_v, [d16 * 2 + j], pexp)
            return c2
        lax.fori_loop(0, BLK1 // L, _grp, 0)

        for j in range(2):
            pltpu.sync_copy(p_v.at[pl.ds(j * BLK1, BLK1)],
                            p_hbm.at[pl.ds(h0 + j * N_EDGES + off, BLK1)])
        return carry
    lax.fori_loop(0, NB1, _block, 0)

    pltpu.sync_copy(den_v, dpart_hbm.at[pl.ds(wid * N_NODES * 2,
                                              N_NODES * 2)])


def _pass1(src, dst, an_sp, ae_t, mb):
    f = functools.partial(
        pl.kernel,
        out_type=(
            jax.ShapeDtypeStruct((HEADS * N_EDGES,), _f32),
            jax.ShapeDtypeStruct((NW * N_NODES * 2,), _f32),
        ),
        mesh=plsc.VectorSubcoreMesh(core_axis_name="c", subcore_axis_name="s",
                                    num_cores=NC, num_subcores=NS),
        compiler_params=pltpu.CompilerParams(needs_layout_passes=False),
        scratch_types=[
            pltpu.VMEM((N_NODES * 4,), _f32),
            pltpu.VMEM((N_NODES * 2,), _f32),
            pltpu.VMEM((BLK1,), _i32),
            pltpu.VMEM((BLK1,), _i32),
            pltpu.VMEM((BLK1 * 2,), _f32),
            pltpu.VMEM((BLK1 * 2,), _f32),
            pltpu.VMEM((2 * L,), _f32),
            pltpu.SemaphoreType.DMA,
        ],
    )
    return f(_pass1_body)(src, dst, an_sp, ae_t, mb)


def _dinv_body(dp_ref, o_ref):
    s = jnp.sum(dp_ref[...], axis=0, keepdims=True)
    o_ref[...] = 1.0 / s


def _dinv(dparts):
    return pl.pallas_call(
        _dinv_body,
        out_shape=jax.ShapeDtypeStruct((1, NC * N_NODES * 2), _f32),
    )(dparts)


def _pass2_body(src_hbm, dst_hbm, p_hbm, h_hbm, eat_hbm,
                out1_hbm, u_hbm,
                sidx_v, didx_v, didx2_v, didx4_v, par_v, q_v,
                p_v, eat_v, hrow_v, msgh_v, msge_v,
                spmh, spme, sem, sem2):
    cid = lax.axis_index("c")
    sid = lax.axis_index("s")
    base = sid * EPW2
    h0 = 2 * cid * N_EDGES

    row1 = sid * RPT
    row2 = sid * RPT2

    def _zmsg(e, carry):
        for c in range(HIDDEN // L):
            msgh_v[e, pl.ds(c * L, L)] = jnp.zeros((L,), _f32)
        return carry
    lax.fori_loop(0, BLK2, _zmsg, 0)

    def _zspm1(j, carry):
        pltpu.sync_copy(msgh_v.at[pl.ds(0, ZCH)],
                        spmh.at[pl.ds(row1 + j * ZCH, ZCH)])
        return carry
    lax.fori_loop(0, RPT // ZCH, _zspm1, 0)

    def _zspm2(j, carry):
        pltpu.sync_copy(msgh_v.at[pl.ds(0, ZCH)],
                        spme.at[pl.ds(row2 + j * ZCH, ZCH)])
        return carry
    lax.fori_loop(0, RPT2 // ZCH, _zspm2, 0)
    plsc.subcore_barrier()

    def _block(i, carry):
        off = base + i * BLK2
        dmas = [
            pltpu.make_async_copy(src_hbm.at[pl.ds(off, BLK2)], sidx_v, sem),
            pltpu.make_async_copy(dst_hbm.at[pl.ds(off, BLK2)], didx_v, sem),
            pltpu.make_async_copy(
                eat_hbm.at[pl.ds(off * EDGE_DIM, BLK2 * EDGE_DIM)], eat_v, sem),
            pltpu.make_async_copy(p_hbm.at[pl.ds(h0 + off, BLK2)],
                                  p_v.at[pl.ds(0, BLK2)], sem),
            pltpu.make_async_copy(p_hbm.at[pl.ds(h0 + N_EDGES + off, BLK2)],
                                  p_v.at[pl.ds(BLK2, BLK2)], sem),
        ]
        for d in dmas:
            d.start()
        for d in dmas:
            d.wait()

        gaths = [
            pltpu.make_async_copy(h_hbm.at[sidx_v.at[pl.ds(c * CH1, CH1)]],
                                  hrow_v.at[pl.ds(c * CH1, CH1)], sem2)
            for c in range(BLK2 // CH1)
        ]
        for g in gaths:
            g.start()

        def _prep(g, c2):
            d16 = didx_v[pl.ds(g * L, L)]
            didx2_v[g // (CH1 // L), pl.ds((g % (CH1 // L)) * L, L)] = (
                lax.shift_right_logical(d16, 1))
            didx4_v[g // (CH1 // L), pl.ds((g % (CH1 // L)) * L, L)] = (
                lax.shift_right_logical(d16, 2))
            par_v[pl.ds(g * L, L)] = (d16 & 1).astype(_f32)
            q_v[pl.ds(g * L, L)] = d16 & 3
            return c2
        lax.fori_loop(0, BLK2 // L, _prep, 0)

        for g in gaths:
            g.wait()

        def _edge(e, c2):
            eidx = jnp.broadcast_to(e, (L,))
            pb = [plsc.load_gather(p_v, [eidx + (j * BLK2)]) for j in range(2)]
            par = plsc.load_gather(par_v, [eidx])
            npar = 1.0 - par
            for k in range(64 // L):
                hv = hrow_v[e, pl.ds(cid * 64 + k * L, L)]
                v = hv * pb[k // 2]
                msgh_v[e, pl.ds(k * L, L)] = v * npar
                msgh_v[e, pl.ds(64 + k * L, L)] = v * par
            qb = plsc.load_gather(q_v, [eidx])
            eav = eat_v[pl.ds(e * EDGE_DIM, EDGE_DIM)]
            mq = [jnp.where(qb == q, jnp.float32(1.0), jnp.float32(0.0))
                  for q in range(4)]
            for j in range(2):
                v = eav * pb[j]
                for q in range(4):
                    msge_v[e, pl.ds(q * 2 * EDGE_DIM + j * EDGE_DIM,
                                    EDGE_DIM)] = v * mq[q]
            return c2
        lax.fori_loop(0, BLK2, _edge, 0)

        for c in range(BLK2 // CH1):
            pltpu.sync_copy(msgh_v.at[pl.ds(c * CH1, CH1)],
                            spmh.at[didx2_v.at[c]], add=True)
            pltpu.sync_copy(msge_v.at[pl.ds(c * CH1, CH1)],
                            spme.at[didx4_v.at[c]], add=True)
        return carry
    lax.fori_loop(0, NB2, _block, 0)
    plsc.subcore_barrier()

    def _flush1(j, carry):
        pltpu.sync_copy(spmh.at[pl.ds(row1 + j * ZCH, ZCH)],
                        out1_hbm.at[pl.ds(cid * SPMH_R + row1 + j * ZCH, ZCH)])
        return carry
    lax.fori_loop(0, RPT // ZCH, _flush1, 0)

    def _flush2(j, carry):
        pltpu.sync_copy(spme.at[pl.ds(row2 + j * ZCH, ZCH)],
                        u_hbm.at[pl.ds(cid * SPME_R + row2 + j * ZCH, ZCH)])
        return carry
    lax.fori_loop(0, RPT2 // ZCH, _flush2, 0)


def _pass2(src, dst, p, h, ea_flat):
    f = functools.partial(
        pl.kernel,
        out_type=(
            jax.ShapeDtypeStruct((NC * SPMH_R, HIDDEN), _f32),
            jax.ShapeDtypeStruct((NC * SPME_R, HIDDEN), _f32),
        ),
        mesh=plsc.VectorSubcoreMesh(core_axis_name="c", subcore_axis_name="s",
                                    num_cores=NC, num_subcores=NS),
        compiler_params=pltpu.CompilerParams(needs_layout_passes=False),
        scratch_types=[
            pltpu.VMEM((BLK2,), _i32),
            pltpu.VMEM((BLK2,), _i32),
            pltpu.VMEM((BLK2 // CH1, CH1), _i32),
            pltpu.VMEM((BLK2 // CH1, CH1), _i32),
            pltpu.VMEM((BLK2,), _f32),
            pltpu.VMEM((BLK2,), _i32),
            pltpu.VMEM((BLK2 * 2,), _f32),
            pltpu.VMEM((BLK2 * EDGE_DIM,), _f32),
            pltpu.VMEM((BLK2, HIDDEN), _f32),
            pltpu.VMEM((BLK2, HIDDEN), _f32),
            pltpu.VMEM((BLK2, HIDDEN), _f32),
            pltpu.VMEM_SHARED((SPMH_R, HIDDEN), _f32),
            pltpu.VMEM_SHARED((SPME_R, HIDDEN), _f32),
            pltpu.SemaphoreType.DMA,
            pltpu.SemaphoreType.DMA,
        ],
    )
    return f(_pass2_body)(src, dst, p, h, ea_flat)


def _final_body(acc1_ref, u_ref, din_ref, nf_ref, we_ref, g_ref, b_ref, o_ref):
    a1 = acc1_ref[...]
    din = din_ref[...]
    halves = []
    for c in range(NC):
        for j in range(2):
            halves.append(a1[c][:, j * HEAD_DIM:(j + 1) * HEAD_DIM]
                          * din[c][:, j][:, None])
    out1 = jnp.concatenate(halves, axis=1)
    u = u_ref[...]
    din = din_ref[...]
    o2 = []
    for h in range(HEADS):
        c, j = divmod(h, 2)
        uh = u[c][:, j * EDGE_DIM:(j + 1) * EDGE_DIM]
        bh = uh * din[c][:, j][:, None]
        wh = we_ref[pl.ds(h * HEAD_DIM, HEAD_DIM), :]
        o2.append(lax.dot_general(bh, wh, (((1,), (1,)), ((), ())),
                                  preferred_element_type=_f32))
    out2 = jnp.concatenate(o2, axis=1)
    pre = out1 + out2 + nf_ref[...]
    mu = jnp.mean(pre, axis=1, keepdims=True)
    cen = pre - mu
    var = jnp.mean(cen * cen, axis=1, keepdims=True)
    normed = cen * lax.rsqrt(var + 1e-5) * g_ref[...] + b_ref[...]
    o_ref[...] = jnp.where(normed > 0, normed, jnp.exp(jnp.minimum(normed, 0.0)) - 1.0)


_BR = 2000


def _final(acc1, u2, din3, node_feat, w_edge, gamma, beta):
    return pl.pallas_call(
        _final_body,
        grid=(N_NODES // _BR,),
        in_specs=[
            pl.BlockSpec((NC, _BR, 64), lambda i: (0, i, 0)),
            pl.BlockSpec((NC, _BR, 2 * EDGE_DIM), lambda i: (0, i, 0)),
            pl.BlockSpec((NC, _BR, 2), lambda i: (0, i, 0)),
            pl.BlockSpec((_BR, HIDDEN), lambda i: (i, 0)),
            pl.BlockSpec((HIDDEN, EDGE_DIM), lambda i: (0, 0)),
            pl.BlockSpec((1, HIDDEN), lambda i: (0, 0)),
            pl.BlockSpec((1, HIDDEN), lambda i: (0, 0)),
        ],
        out_specs=pl.BlockSpec((_BR, HIDDEN), lambda i: (i, 0)),
        out_shape=jax.ShapeDtypeStruct((N_NODES, HIDDEN), _f32),
    )(acc1, u2, din3, node_feat, w_edge, gamma, beta)


def kernel(node_feat, edge_attr, W_node, W_edge, att_src, att_dst, att_edge,
           ln_gamma, ln_beta, edge_index):
    src = edge_index[0].astype(_i32)
    dst = edge_index[1].astype(_i32)

    eye = jnp.eye(HEADS, dtype=_f32)
    a_src_m = (eye[:, None, :] * att_src[:, :, None]).reshape(HIDDEN, HEADS)
    a_dst_m = (eye[:, None, :] * att_dst[:, :, None]).reshape(HIDDEN, HEADS)
    a_edge_m = (eye[:, None, :] * att_edge[:, :, None]).reshape(HIDDEN, HEADS)
    a_sd = jnp.concatenate([a_src_m, a_dst_m], axis=1)

    h, a_nodes, m_node = _node_proj(node_feat, W_node, a_sd)
    ae_t, m_edge = _edge_coef(edge_attr.T, W_edge, a_edge_m)

    mb = m_node[0, :HEADS] + m_node[0, HEADS:] + m_edge[:, 0]
    mb = jnp.where(mb >= 0.0, mb, mb * 0.2)
    mb16 = jnp.broadcast_to(mb[:, None], (HEADS, L)).reshape(-1)

    asrc, adst = a_nodes[:, :HEADS], a_nodes[:, HEADS:]
    an_sp = jnp.concatenate(
        [jnp.concatenate([asrc[:, 2 * c:2 * c + 2], adst[:, 2 * c:2 * c + 2]],
                         axis=1).reshape(-1) for c in range(NC)])

    p, dparts = _pass1(src, dst, an_sp, ae_t.reshape(-1), mb16)
    dinv = _dinv(dparts.reshape(NS, NC * N_NODES * 2)).reshape(-1)
    acc1, u = _pass2(src, dst, p, h, edge_attr.reshape(-1))
    a1 = acc1.reshape(NC, SPMH_R * 2, 64)[:, :N_NODES, :]
    u2 = u.reshape(NC, SPME_R * 4, 2 * EDGE_DIM)[:, :N_NODES, :]
    return _final(a1, u2,
                  dinv.reshape(NC, N_NODES, 2), node_feat, W_edge,
                  ln_gamma.reshape(1, HIDDEN), ln_beta.reshape(1, HIDDEN))


# speedup vs baseline: 34.0051x; 1.4690x over previous
import functools

import jax
import jax.numpy as jnp
from jax import lax
from jax.experimental import pallas as pl
from jax.experimental.pallas import tpu as pltpu
from jax.experimental.pallas import tpu_sc as plsc

N_NODES = 10000
N_EDGES = 320000
NODE_DIM = 128
EDGE_DIM = 16
HIDDEN = 128
HEADS = 4
HEAD_DIM = HIDDEN // HEADS

NC = 2
NS = 16
NW = NC * NS
L = 16

EPW1 = N_EDGES // NS
BLK1 = 160
NB1 = EPW1 // BLK1
CH1 = 80
EPW2 = N_EDGES // NS
BLK2 = 160
NB2 = EPW2 // BLK2
SPMH_R = 5120
RPT = SPMH_R // NS
ZCH = 16
SPME_R = 2560
RPT2 = SPME_R // NS

_f32 = jnp.float32
_i32 = jnp.int32


def _node_proj_body(nf_ref, wn_ref, asd_ref, h_ref, a_ref, m_ref):
    h = lax.dot_general(nf_ref[...], wn_ref[...],
                        (((1,), (1,)), ((), ())),
                        preferred_element_type=_f32)
    h_ref[...] = h
    a = lax.dot_general(h, asd_ref[...], (((1,), (0,)), ((), ())),
                        preferred_element_type=_f32)
    a_ref[...] = a
    m_ref[...] = jnp.broadcast_to(jnp.max(a, axis=0, keepdims=True), (8, 8))


def _node_proj(node_feat, w_node, a_sd):
    return pl.pallas_call(
        _node_proj_body,
        out_shape=(
            jax.ShapeDtypeStruct((N_NODES, HIDDEN), _f32),
            jax.ShapeDtypeStruct((N_NODES, 8), _f32),
            jax.ShapeDtypeStruct((8, 8), _f32),
        ),
    )(node_feat, w_node, a_sd)


def _edge_coef_body(eat_ref, we_ref, ae_ref, o_ref, m_ref):
    c = lax.dot_general(we_ref[...], ae_ref[...], (((0,), (0,)), ((), ())),
                        preferred_element_type=_f32)
    o = lax.dot_general(c, eat_ref[...], (((0,), (0,)), ((), ())),
                        preferred_element_type=_f32)
    o_ref[...] = o
    m_ref[...] = jnp.broadcast_to(jnp.max(o, axis=1, keepdims=True), (HEADS, 8))


def _edge_coef(edge_attr_t, w_edge, a_edge):
    return pl.pallas_call(
        _edge_coef_body,
        out_shape=(
            jax.ShapeDtypeStruct((HEADS, N_EDGES), _f32),
            jax.ShapeDtypeStruct((HEADS, 8), _f32),
        ),
    )(edge_attr_t, w_edge, a_edge)


def _pass1_body(src_hbm, dst_hbm, an_hbm, ae_hbm, mb_hbm,
                p_hbm, dpart_hbm,
                an_v, den_v, src_v, dst_v, ae_v, p_v, mb_v, sem):
    cid = lax.axis_index("c")
    sid = lax.axis_index("s")
    wid = sid * NC + cid
    base = sid * EPW1
    h0 = 2 * cid * N_EDGES

    pltpu.sync_copy(an_hbm.at[pl.ds(cid * N_NODES * 4, N_NODES * 4)], an_v)
    pltpu.sync_copy(mb_hbm.at[pl.ds(cid * 2 * L, 2 * L)], mb_v)
    mb_b = [mb_v[pl.ds(j * L, L)] for j in range(2)]

    def _zero(i, carry):
        den_v[pl.ds(i * L, L)] = jnp.zeros((L,), _f32)
        return carry
    lax.fori_loop(0, N_NODES * 2 // L, _zero, 0)

    def _block(i, carry):
        off = base + i * BLK1
        pltpu.sync_copy(src_hbm.at[pl.ds(off, BLK1)], src_v)
        pltpu.sync_copy(dst_hbm.at[pl.ds(off, BLK1)], dst_v)
        for j in range(2):
            pltpu.sync_copy(ae_hbm.at[pl.ds(h0 + j * N_EDGES + off, BLK1)],
                            ae_v.at[pl.ds(j * BLK1, BLK1)])

        def _grp(g, c2):
            s16 = src_v[pl.ds(g * L, L)]
            d16 = dst_v[pl.ds(g * L, L)]
            for j in range(2):
                asrc = plsc.load_gather(an_v, [s16 * 4 + j])
                adst = plsc.load_gather(an_v, [d16 * 4 + (2 + j)])
                ae16 = ae_v[pl.ds(j * BLK1 + g * L, L)]
                lg = asrc + adst + ae16
                lg = jnp.where(lg >= 0.0, lg, lg * jnp.float32(0.2))
                pexp = jnp.exp(lg - mb_b[j])
                p_v[pl.ds(j * BLK1 + g * L, L)] = pexp
                plsc.addupdate_scatter(den_v, [d16 * 2 + j], pexp)
            return c2
        lax.fori_loop(0, BLK1 // L, _grp, 0)

        for j in range(2):
            pltpu.sync_copy(p_v.at[pl.ds(j * BLK1, BLK1)],
                            p_hbm.at[pl.ds(h0 + j * N_EDGES + off, BLK1)])
        return carry
    lax.fori_loop(0, NB1, _block, 0)

    pltpu.sync_copy(den_v, dpart_hbm.at[pl.ds(wid * N_NODES * 2,
                                              N_NODES * 2)])


def _pass1(src, dst, an_sp, ae_t, mb):
    f = functools.partial(
        pl.kernel,
        out_type=(
            jax.ShapeDtypeStruct((HEADS * N_EDGES,), _f32),
            jax.ShapeDtypeStruct((NW * N_NODES * 2,), _f32),
        ),
        mesh=plsc.VectorSubcoreMesh(core_axis_name="c", subcore_axis_name="s",
                                    num_cores=NC, num_subcores=NS),
        compiler_params=pltpu.CompilerParams(needs_layout_passes=False),
        scratch_types=[
            pltpu.VMEM((N_NODES * 4,), _f32),
            pltpu.VMEM((N_NODES * 2,), _f32),
            pltpu.VMEM((BLK1,), _i32),
            pltpu.VMEM((BLK1,), _i32),
            pltpu.VMEM((BLK1 * 2,), _f32),
            pltpu.VMEM((BLK1 * 2,), _f32),
            pltpu.VMEM((2 * L,), _f32),
            pltpu.SemaphoreType.DMA,
        ],
    )
    return f(_pass1_body)(src, dst, an_sp, ae_t, mb)


def _dinv_body(dp_ref, o_ref):
    s = jnp.sum(dp_ref[...], axis=0, keepdims=True)
    o_ref[...] = 1.0 / s


def _dinv(dparts):
    return pl.pallas_call(
        _dinv_body,
        out_shape=jax.ShapeDtypeStruct((1, NC * N_NODES * 2), _f32),
    )(dparts)


def _pass2_body(src_hbm, dst_hbm, p_hbm, h_hbm, eat_hbm,
                out1_hbm, u_hbm,
                sidx_v, didx_v, didx2_v, didx4_v, par_v, q_v,
                p_v, eat_v, hrow_v, msgh_v, msge_v,
                spmh, spme, sem, sem2):
    cid = lax.axis_index("c")
    sid = lax.axis_index("s")
    base = sid * EPW2
    h0 = 2 * cid * N_EDGES

    row1 = sid * RPT
    row2 = sid * RPT2

    def _zmsg(e, carry):
        for c in range(HIDDEN // L):
            msgh_v[e, pl.ds(c * L, L)] = jnp.zeros((L,), _f32)
        return carry
    lax.fori_loop(0, BLK2, _zmsg, 0)

    def _zspm1(j, carry):
        pltpu.sync_copy(msgh_v.at[pl.ds(0, ZCH)],
                        spmh.at[pl.ds(row1 + j * ZCH, ZCH)])
        return carry
    lax.fori_loop(0, RPT // ZCH, _zspm1, 0)

    def _zspm2(j, carry):
        pltpu.sync_copy(msgh_v.at[pl.ds(0, ZCH)],
                        spme.at[pl.ds(row2 + j * ZCH, ZCH)])
        return carry
    lax.fori_loop(0, RPT2 // ZCH, _zspm2, 0)
    plsc.subcore_barrier()

    def _block(i, carry):
        off = base + i * BLK2
        dmas = [
            pltpu.make_async_copy(src_hbm.at[pl.ds(off, BLK2)], sidx_v, sem),
            pltpu.make_async_copy(dst_hbm.at[pl.ds(off, BLK2)], didx_v, sem),
            pltpu.make_async_copy(
                eat_hbm.at[pl.ds(off * EDGE_DIM, BLK2 * EDGE_DIM)], eat_v, sem),
            pltpu.make_async_copy(p_hbm.at[pl.ds(h0 + off, BLK2)],
                                  p_v.at[pl.ds(0, BLK2)], sem),
            pltpu.make_async_copy(p_hbm.at[pl.ds(h0 + N_EDGES + off, BLK2)],
                                  p_v.at[pl.ds(BLK2, BLK2)], sem),
        ]
        for d in dmas:
            d.start()
        for d in dmas:
            d.wait()

        gaths = [
            pltpu.make_async_copy(h_hbm.at[sidx_v.at[pl.ds(c * CH1, CH1)]],
                                  hrow_v.at[pl.ds(c * CH1, CH1)], sem2)
            for c in range(BLK2 // CH1)
        ]
        for g in gaths:
            g.start()

        @plsc.parallel_loop(0, BLK2 // L, unroll=2)
        def _prep(g):
            d16 = didx_v[pl.ds(g * L, L)]
            didx2_v[g // (CH1 // L), pl.ds((g % (CH1 // L)) * L, L)] = (
                lax.shift_right_logical(d16, 1))
            didx4_v[g // (CH1 // L), pl.ds((g % (CH1 // L)) * L, L)] = (
                lax.shift_right_logical(d16, 2))
            par_v[pl.ds(g * L, L)] = (d16 & 1).astype(_f32)
            q_v[pl.ds(g * L, L)] = d16 & 3

        for g in gaths:
            g.wait()

        @plsc.parallel_loop(0, BLK2, unroll=2)
        def _edge(e):
            eidx = jnp.broadcast_to(e, (L,))
            pb = [plsc.load_gather(p_v, [eidx + (j * BLK2)]) for j in range(2)]
            par = plsc.load_gather(par_v, [eidx])
            npar = 1.0 - par
            for k in range(64 // L):
                hv = hrow_v[e, pl.ds(cid * 64 + k * L, L)]
                v = hv * pb[k // 2]
                msgh_v[e, pl.ds(k * L, L)] = v * npar
                msgh_v[e, pl.ds(64 + k * L, L)] = v * par
            qb = plsc.load_gather(q_v, [eidx])
            eav = eat_v[pl.ds(e * EDGE_DIM, EDGE_DIM)]
            mq = [jnp.where(qb == q, jnp.float32(1.0), jnp.float32(0.0))
                  for q in range(4)]
            for j in range(2):
                v = eav * pb[j]
                for q in range(4):
                    msge_v[e, pl.ds(q * 2 * EDGE_DIM + j * EDGE_DIM,
                                    EDGE_DIM)] = v * mq[q]

        for c in range(BLK2 // CH1):
            pltpu.sync_copy(msgh_v.at[pl.ds(c * CH1, CH1)],
                            spmh.at[didx2_v.at[c]], add=True)
            pltpu.sync_copy(msge_v.at[pl.ds(c * CH1, CH1)],
                            spme.at[didx4_v.at[c]], add=True)
        return carry
    lax.fori_loop(0, NB2, _block, 0)
    plsc.subcore_barrier()

    def _flush1(j, carry):
        pltpu.sync_copy(spmh.at[pl.ds(row1 + j * ZCH, ZCH)],
                        out1_hbm.at[pl.ds(cid * SPMH_R + row1 + j * ZCH, ZCH)])
        return carry
    lax.fori_loop(0, RPT // ZCH, _flush1, 0)

    def _flush2(j, carry):
        pltpu.sync_copy(spme.at[pl.ds(row2 + j * ZCH, ZCH)],
                        u_hbm.at[pl.ds(cid * SPME_R + row2 + j * ZCH, ZCH)])
        return carry
    lax.fori_loop(0, RPT2 // ZCH, _flush2, 0)


def _pass2(src, dst, p, h, ea_flat):
    f = functools.partial(
        pl.kernel,
        out_type=(
            jax.ShapeDtypeStruct((NC * SPMH_R, HIDDEN), _f32),
            jax.ShapeDtypeStruct((NC * SPME_R, HIDDEN), _f32),
        ),
        mesh=plsc.VectorSubcoreMesh(core_axis_name="c", subcore_axis_name="s",
                                    num_cores=NC, num_subcores=NS),
        compiler_params=pltpu.CompilerParams(needs_layout_passes=False),
        scratch_types=[
            pltpu.VMEM((BLK2,), _i32),
            pltpu.VMEM((BLK2,), _i32),
            pltpu.VMEM((BLK2 // CH1, CH1), _i32),
            pltpu.VMEM((BLK2 // CH1, CH1), _i32),
            pltpu.VMEM((BLK2,), _f32),
            pltpu.VMEM((BLK2,), _i32),
            pltpu.VMEM((BLK2 * 2,), _f32),
            pltpu.VMEM((BLK2 * EDGE_DIM,), _f32),
            pltpu.VMEM((BLK2, HIDDEN), _f32),
            pltpu.VMEM((BLK2, HIDDEN), _f32),
            pltpu.VMEM((BLK2, HIDDEN), _f32),
            pltpu.VMEM_SHARED((SPMH_R, HIDDEN), _f32),
            pltpu.VMEM_SHARED((SPME_R, HIDDEN), _f32),
            pltpu.SemaphoreType.DMA,
            pltpu.SemaphoreType.DMA,
        ],
    )
    return f(_pass2_body)(src, dst, p, h, ea_flat)


def _final_body(acc1_ref, u_ref, din_ref, nf_ref, we_ref, g_ref, b_ref, o_ref):
    a1 = acc1_ref[...]
    din = din_ref[...]
    halves = []
    for c in range(NC):
        for j in range(2):
            halves.append(a1[c][:, j * HEAD_DIM:(j + 1) * HEAD_DIM]
                          * din[c][:, j][:, None])
    out1 = jnp.concatenate(halves, axis=1)
    u = u_ref[...]
    din = din_ref[...]
    o2 = []
    for h in range(HEADS):
        c, j = divmod(h, 2)
        uh = u[c][:, j * EDGE_DIM:(j + 1) * EDGE_DIM]
        bh = uh * din[c][:, j][:, None]
        wh = we_ref[pl.ds(h * HEAD_DIM, HEAD_DIM), :]
        o2.append(lax.dot_general(bh, wh, (((1,), (1,)), ((), ())),
                                  preferred_element_type=_f32))
    out2 = jnp.concatenate(o2, axis=1)
    pre = out1 + out2 + nf_ref[...]
    mu = jnp.mean(pre, axis=1, keepdims=True)
    cen = pre - mu
    var = jnp.mean(cen * cen, axis=1, keepdims=True)
    normed = cen * lax.rsqrt(var + 1e-5) * g_ref[...] + b_ref[...]
    o_ref[...] = jnp.where(normed > 0, normed, jnp.exp(jnp.minimum(normed, 0.0)) - 1.0)


_BR = 2000


def _final(acc1, u2, din3, node_feat, w_edge, gamma, beta):
    return pl.pallas_call(
        _final_body,
        grid=(N_NODES // _BR,),
        in_specs=[
            pl.BlockSpec((NC, _BR, 64), lambda i: (0, i, 0)),
            pl.BlockSpec((NC, _BR, 2 * EDGE_DIM), lambda i: (0, i, 0)),
            pl.BlockSpec((NC, _BR, 2), lambda i: (0, i, 0)),
            pl.BlockSpec((_BR, HIDDEN), lambda i: (i, 0)),
            pl.BlockSpec((HIDDEN, EDGE_DIM), lambda i: (0, 0)),
            pl.BlockSpec((1, HIDDEN), lambda i: (0, 0)),
            pl.BlockSpec((1, HIDDEN), lambda i: (0, 0)),
        ],
        out_specs=pl.BlockSpec((_BR, HIDDEN), lambda i: (i, 0)),
        out_shape=jax.ShapeDtypeStruct((N_NODES, HIDDEN), _f32),
    )(acc1, u2, din3, node_feat, w_edge, gamma, beta)


def kernel(node_feat, edge_attr, W_node, W_edge, att_src, att_dst, att_edge,
           ln_gamma, ln_beta, edge_index):
    src = edge_index[0].astype(_i32)
    dst = edge_index[1].astype(_i32)

    eye = jnp.eye(HEADS, dtype=_f32)
    a_src_m = (eye[:, None, :] * att_src[:, :, None]).reshape(HIDDEN, HEADS)
    a_dst_m = (eye[:, None, :] * att_dst[:, :, None]).reshape(HIDDEN, HEADS)
    a_edge_m = (eye[:, None, :] * att_edge[:, :, None]).reshape(HIDDEN, HEADS)
    a_sd = jnp.concatenate([a_src_m, a_dst_m], axis=1)

    h, a_nodes, m_node = _node_proj(node_feat, W_node, a_sd)
    ae_t, m_edge = _edge_coef(edge_attr.T, W_edge, a_edge_m)

    mb = m_node[0, :HEADS] + m_node[0, HEADS:] + m_edge[:, 0]
    mb = jnp.where(mb >= 0.0, mb, mb * 0.2)
    mb16 = jnp.broadcast_to(mb[:, None], (HEADS, L)).reshape(-1)

    asrc, adst = a_nodes[:, :HEADS], a_nodes[:, HEADS:]
    an_sp = jnp.concatenate(
        [jnp.concatenate([asrc[:, 2 * c:2 * c + 2], adst[:, 2 * c:2 * c + 2]],
                         axis=1).reshape(-1) for c in range(NC)])

    p, dparts = _pass1(src, dst, an_sp, ae_t.reshape(-1), mb16)
    dinv = _dinv(dparts.reshape(NS, NC * N_NODES * 2)).reshape(-1)
    acc1, u = _pass2(src, dst, p, h, edge_attr.reshape(-1))
    a1 = acc1.reshape(NC, SPMH_R * 2, 64)[:, :N_NODES, :]
    u2 = u.reshape(NC, SPME_R * 4, 2 * EDGE_DIM)[:, :N_NODES, :]
    return _final(a1, u2,
                  dinv.reshape(NC, N_NODES, 2), node_feat, W_edge,
                  ln_gamma.reshape(1, HIDDEN), ln_beta.reshape(1, HIDDEN))


# speedup vs baseline: 39.4531x; 1.1602x over previous
import functools

import jax
import jax.numpy as jnp
from jax import lax
from jax.experimental import pallas as pl
from jax.experimental.pallas import tpu as pltpu
from jax.experimental.pallas import tpu_sc as plsc

N_NODES = 10000
N_EDGES = 320000
NODE_DIM = 128
EDGE_DIM = 16
HIDDEN = 128
HEADS = 4
HEAD_DIM = HIDDEN // HEADS

NC = 2
NS = 16
NW = NC * NS
L = 16

EPW1 = N_EDGES // NS
BLK1 = 160
NB1 = EPW1 // BLK1
CH1 = 80
EPW2 = N_EDGES // NS
BLK2 = 160
NB2 = EPW2 // BLK2
SPMH_R = 5120
RPT = SPMH_R // NS
ZCH = 16
SPME_R = 2560
RPT2 = SPME_R // NS

_f32 = jnp.float32
_i32 = jnp.int32


def _node_proj_body(nf_ref, wn_ref, asd_ref, h_ref, a_ref, m_ref):
    h = lax.dot_general(nf_ref[...], wn_ref[...],
                        (((1,), (1,)), ((), ())),
                        preferred_element_type=_f32)
    h_ref[...] = h
    a = lax.dot_general(h, asd_ref[...], (((1,), (0,)), ((), ())),
                        preferred_element_type=_f32)
    a_ref[...] = a
    m_ref[...] = jnp.broadcast_to(jnp.max(a, axis=0, keepdims=True), (8, 8))


def _node_proj(node_feat, w_node, a_sd):
    return pl.pallas_call(
        _node_proj_body,
        out_shape=(
            jax.ShapeDtypeStruct((N_NODES, HIDDEN), _f32),
            jax.ShapeDtypeStruct((N_NODES, 8), _f32),
            jax.ShapeDtypeStruct((8, 8), _f32),
        ),
    )(node_feat, w_node, a_sd)


def _edge_coef_body(eat_ref, we_ref, ae_ref, o_ref, m_ref):
    c = lax.dot_general(we_ref[...], ae_ref[...], (((0,), (0,)), ((), ())),
                        preferred_element_type=_f32)
    o = lax.dot_general(c, eat_ref[...], (((0,), (0,)), ((), ())),
                        preferred_element_type=_f32)
    o_ref[...] = o
    m_ref[...] = jnp.broadcast_to(jnp.max(o, axis=1, keepdims=True), (HEADS, 8))


def _edge_coef(edge_attr_t, w_edge, a_edge):
    return pl.pallas_call(
        _edge_coef_body,
        out_shape=(
            jax.ShapeDtypeStruct((HEADS, N_EDGES), _f32),
            jax.ShapeDtypeStruct((HEADS, 8), _f32),
        ),
    )(edge_attr_t, w_edge, a_edge)


def _pass1_body(src_hbm, dst_hbm, an_hbm, ae_hbm, mb_hbm,
                p_hbm, dpart_hbm,
                an_v, den_v, src_v, dst_v, ae_v, p_v, mb_v, sem):
    cid = lax.axis_index("c")
    sid = lax.axis_index("s")
    wid = sid * NC + cid
    base = sid * EPW1
    h0 = 2 * cid * N_EDGES

    pltpu.sync_copy(an_hbm.at[pl.ds(cid * N_NODES * 4, N_NODES * 4)], an_v)
    pltpu.sync_copy(mb_hbm.at[pl.ds(cid * 2 * L, 2 * L)], mb_v)
    mb_b = [mb_v[pl.ds(j * L, L)] for j in range(2)]

    def _zero(i, carry):
        den_v[pl.ds(i * L, L)] = jnp.zeros((L,), _f32)
        return carry
    lax.fori_loop(0, N_NODES * 2 // L, _zero, 0)

    def _block(i, carry):
        off = base + i * BLK1
        dmas = [
            pltpu.make_async_copy(src_hbm.at[pl.ds(off, BLK1)], src_v, sem),
            pltpu.make_async_copy(dst_hbm.at[pl.ds(off, BLK1)], dst_v, sem),
            pltpu.make_async_copy(ae_hbm.at[pl.ds(h0 + off, BLK1)],
                                  ae_v.at[pl.ds(0, BLK1)], sem),
            pltpu.make_async_copy(ae_hbm.at[pl.ds(h0 + N_EDGES + off, BLK1)],
                                  ae_v.at[pl.ds(BLK1, BLK1)], sem),
        ]
        for d in dmas:
            d.start()
        for d in dmas:
            d.wait()

        def _grp(g, c2):
            s16 = src_v[pl.ds(g * L, L)]
            d16 = dst_v[pl.ds(g * L, L)]
            for j in range(2):
                asrc = plsc.load_gather(an_v, [s16 * 4 + j])
                adst = plsc.load_gather(an_v, [d16 * 4 + (2 + j)])
                ae16 = ae_v[pl.ds(j * BLK1 + g * L, L)]
                lg = asrc + adst + ae16
                lg = jnp.where(lg >= 0.0, lg, lg * jnp.float32(0.2))
                pexp = jnp.exp(lg - mb_b[j])
                p_v[pl.ds(j * BLK1 + g * L, L)] = pexp
                plsc.addupdate_scatter(den_v, [d16 * 2 + j], pexp)
            return c2
        lax.fori_loop(0, BLK1 // L, _grp, 0)

        for j in range(2):
            pltpu.sync_copy(p_v.at[pl.ds(j * BLK1, BLK1)],
                            p_hbm.at[pl.ds(h0 + j * N_EDGES + off, BLK1)])
        return carry
    lax.fori_loop(0, NB1, _block, 0)

    pltpu.sync_copy(den_v, dpart_hbm.at[pl.ds(wid * N_NODES * 2,
                                              N_NODES * 2)])


def _pass1(src, dst, an_sp, ae_t, mb):
    f = functools.partial(
        pl.kernel,
        out_type=(
            jax.ShapeDtypeStruct((HEADS * N_EDGES,), _f32),
            jax.ShapeDtypeStruct((NW * N_NODES * 2,), _f32),
        ),
        mesh=plsc.VectorSubcoreMesh(core_axis_name="c", subcore_axis_name="s",
                                    num_cores=NC, num_subcores=NS),
        compiler_params=pltpu.CompilerParams(needs_layout_passes=False),
        scratch_types=[
            pltpu.VMEM((N_NODES * 4,), _f32),
            pltpu.VMEM((N_NODES * 2,), _f32),
            pltpu.VMEM((BLK1,), _i32),
            pltpu.VMEM((BLK1,), _i32),
            pltpu.VMEM((BLK1 * 2,), _f32),
            pltpu.VMEM((BLK1 * 2,), _f32),
            pltpu.VMEM((2 * L,), _f32),
            pltpu.SemaphoreType.DMA,
        ],
    )
    return f(_pass1_body)(src, dst, an_sp, ae_t, mb)


def _dinv_body(dp_ref, o_ref):
    s = jnp.sum(dp_ref[...], axis=0, keepdims=True)
    o_ref[...] = 1.0 / s


def _dinv(dparts):
    return pl.pallas_call(
        _dinv_body,
        out_shape=jax.ShapeDtypeStruct((1, NC * N_NODES * 2), _f32),
    )(dparts)


def _pass2_body(src_hbm, dst_hbm, p_hbm, h_hbm, eat_hbm,
                out1_hbm, u_hbm,
                sidx_v, didx_v, didx2_v, didx4_v, par_v, q_v,
                p_v, eat_v, hrow_v, msgh_v, msge_v,
                spmh, spme, sem, sem2):
    cid = lax.axis_index("c")
    sid = lax.axis_index("s")
    base = sid * EPW2
    h0 = 2 * cid * N_EDGES

    row1 = sid * RPT
    row2 = sid * RPT2

    def _zmsg(e, carry):
        for c in range(HIDDEN // L):
            msgh_v[e, pl.ds(c * L, L)] = jnp.zeros((L,), _f32)
        return carry
    lax.fori_loop(0, BLK2, _zmsg, 0)

    def _zspm1(j, carry):
        pltpu.sync_copy(msgh_v.at[pl.ds(0, ZCH)],
                        spmh.at[pl.ds(row1 + j * ZCH, ZCH)])
        return carry
    lax.fori_loop(0, RPT // ZCH, _zspm1, 0)

    def _zspm2(j, carry):
        pltpu.sync_copy(msgh_v.at[pl.ds(0, ZCH)],
                        spme.at[pl.ds(row2 + j * ZCH, ZCH)])
        return carry
    lax.fori_loop(0, RPT2 // ZCH, _zspm2, 0)
    plsc.subcore_barrier()

    def _block(i, carry):
        off = base + i * BLK2
        dmas = [
            pltpu.make_async_copy(src_hbm.at[pl.ds(off, BLK2)], sidx_v, sem),
            pltpu.make_async_copy(dst_hbm.at[pl.ds(off, BLK2)], didx_v, sem),
            pltpu.make_async_copy(
                eat_hbm.at[pl.ds(off * EDGE_DIM, BLK2 * EDGE_DIM)], eat_v, sem),
            pltpu.make_async_copy(p_hbm.at[pl.ds(h0 + off, BLK2)],
                                  p_v.at[pl.ds(0, BLK2)], sem),
            pltpu.make_async_copy(p_hbm.at[pl.ds(h0 + N_EDGES + off, BLK2)],
                                  p_v.at[pl.ds(BLK2, BLK2)], sem),
        ]
        for d in dmas:
            d.start()
        for d in dmas:
            d.wait()

        gaths = [
            pltpu.make_async_copy(h_hbm.at[sidx_v.at[pl.ds(c * CH1, CH1)]],
                                  hrow_v.at[pl.ds(c * CH1, CH1)], sem2)
            for c in range(BLK2 // CH1)
        ]
        for g in gaths:
            g.start()

        @plsc.parallel_loop(0, BLK2 // L, unroll=2)
        def _prep(g):
            d16 = didx_v[pl.ds(g * L, L)]
            didx2_v[g // (CH1 // L), pl.ds((g % (CH1 // L)) * L, L)] = (
                lax.shift_right_logical(d16, 1))
            didx4_v[g // (CH1 // L), pl.ds((g % (CH1 // L)) * L, L)] = (
                lax.shift_right_logical(d16, 2))
            par_v[pl.ds(g * L, L)] = (d16 & 1).astype(_f32)
            q_v[pl.ds(g * L, L)] = d16 & 3

        for g in gaths:
            g.wait()

        @plsc.parallel_loop(0, BLK2, unroll=4)
        def _edge(e):
            eidx = jnp.broadcast_to(e, (L,))
            pb = [plsc.load_gather(p_v, [eidx + (j * BLK2)]) for j in range(2)]
            par = plsc.load_gather(par_v, [eidx])
            npar = 1.0 - par
            for k in range(64 // L):
                hv = hrow_v[e, pl.ds(cid * 64 + k * L, L)]
                v = hv * pb[k // 2]
                msgh_v[e, pl.ds(k * L, L)] = v * npar
                msgh_v[e, pl.ds(64 + k * L, L)] = v * par
            qb = plsc.load_gather(q_v, [eidx])
            eav = eat_v[pl.ds(e * EDGE_DIM, EDGE_DIM)]
            mq = [jnp.where(qb == q, jnp.float32(1.0), jnp.float32(0.0))
                  for q in range(4)]
            for j in range(2):
                v = eav * pb[j]
                for q in range(4):
                    msge_v[e, pl.ds(q * 2 * EDGE_DIM + j * EDGE_DIM,
                                    EDGE_DIM)] = v * mq[q]

        for c in range(BLK2 // CH1):
            pltpu.sync_copy(msgh_v.at[pl.ds(c * CH1, CH1)],
                            spmh.at[didx2_v.at[c]], add=True)
            pltpu.sync_copy(msge_v.at[pl.ds(c * CH1, CH1)],
                            spme.at[didx4_v.at[c]], add=True)
        return carry
    lax.fori_loop(0, NB2, _block, 0)
    plsc.subcore_barrier()

    def _flush1(j, carry):
        pltpu.sync_copy(spmh.at[pl.ds(row1 + j * ZCH, ZCH)],
                        out1_hbm.at[pl.ds(cid * SPMH_R + row1 + j * ZCH, ZCH)])
        return carry
    lax.fori_loop(0, RPT // ZCH, _flush1, 0)

    def _flush2(j, carry):
        pltpu.sync_copy(spme.at[pl.ds(row2 + j * ZCH, ZCH)],
                        u_hbm.at[pl.ds(cid * SPME_R + row2 + j * ZCH, ZCH)])
        return carry
    lax.fori_loop(0, RPT2 // ZCH, _flush2, 0)


def _pass2(src, dst, p, h, ea_flat):
    f = functools.partial(
        pl.kernel,
        out_type=(
            jax.ShapeDtypeStruct((NC * SPMH_R, HIDDEN), _f32),
            jax.ShapeDtypeStruct((NC * SPME_R, HIDDEN), _f32),
        ),
        mesh=plsc.VectorSubcoreMesh(core_axis_name="c", subcore_axis_name="s",
                                    num_cores=NC, num_subcores=NS),
        compiler_params=pltpu.CompilerParams(needs_layout_passes=False),
        scratch_types=[
            pltpu.VMEM((BLK2,), _i32),
            pltpu.VMEM((BLK2,), _i32),
            pltpu.VMEM((BLK2 // CH1, CH1), _i32),
            pltpu.VMEM((BLK2 // CH1, CH1), _i32),
            pltpu.VMEM((BLK2,), _f32),
            pltpu.VMEM((BLK2,), _i32),
            pltpu.VMEM((BLK2 * 2,), _f32),
            pltpu.VMEM((BLK2 * EDGE_DIM,), _f32),
            pltpu.VMEM((BLK2, HIDDEN), _f32),
            pltpu.VMEM((BLK2, HIDDEN), _f32),
            pltpu.VMEM((BLK2, HIDDEN), _f32),
            pltpu.VMEM_SHARED((SPMH_R, HIDDEN), _f32),
            pltpu.VMEM_SHARED((SPME_R, HIDDEN), _f32),
            pltpu.SemaphoreType.DMA,
            pltpu.SemaphoreType.DMA,
        ],
    )
    return f(_pass2_body)(src, dst, p, h, ea_flat)


def _final_body(acc1_ref, u_ref, din_ref, nf_ref, we_ref, g_ref, b_ref, o_ref):
    a1 = acc1_ref[...]
    din = din_ref[...]
    halves = []
    for c in range(NC):
        for j in range(2):
            halves.append(a1[c][:, j * HEAD_DIM:(j + 1) * HEAD_DIM]
                          * din[c][:, j][:, None])
    out1 = jnp.concatenate(halves, axis=1)
    u = u_ref[...]
    din = din_ref[...]
    o2 = []
    for h in range(HEADS):
        c, j = divmod(h, 2)
        uh = u[c][:, j * EDGE_DIM:(j + 1) * EDGE_DIM]
        bh = uh * din[c][:, j][:, None]
        wh = we_ref[pl.ds(h * HEAD_DIM, HEAD_DIM), :]
        o2.append(lax.dot_general(bh, wh, (((1,), (1,)), ((), ())),
                                  preferred_element_type=_f32))
    out2 = jnp.concatenate(o2, axis=1)
    pre = out1 + out2 + nf_ref[...]
    mu = jnp.mean(pre, axis=1, keepdims=True)
    cen = pre - mu
    var = jnp.mean(cen * cen, axis=1, keepdims=True)
    normed = cen * lax.rsqrt(var + 1e-5) * g_ref[...] + b_ref[...]
    o_ref[...] = jnp.where(normed > 0, normed, jnp.exp(jnp.minimum(normed, 0.0)) - 1.0)


_BR = 2000


def _final(acc1, u2, din3, node_feat, w_edge, gamma, beta):
    return pl.pallas_call(
        _final_body,
        grid=(N_NODES // _BR,),
        in_specs=[
            pl.BlockSpec((NC, _BR, 64), lambda i: (0, i, 0)),
            pl.BlockSpec((NC, _BR, 2 * EDGE_DIM), lambda i: (0, i, 0)),
            pl.BlockSpec((NC, _BR, 2), lambda i: (0, i, 0)),
            pl.BlockSpec((_BR, HIDDEN), lambda i: (i, 0)),
            pl.BlockSpec((HIDDEN, EDGE_DIM), lambda i: (0, 0)),
            pl.BlockSpec((1, HIDDEN), lambda i: (0, 0)),
            pl.BlockSpec((1, HIDDEN), lambda i: (0, 0)),
        ],
        out_specs=pl.BlockSpec((_BR, HIDDEN), lambda i: (i, 0)),
        out_shape=jax.ShapeDtypeStruct((N_NODES, HIDDEN), _f32),
    )(acc1, u2, din3, node_feat, w_edge, gamma, beta)


def kernel(node_feat, edge_attr, W_node, W_edge, att_src, att_dst, att_edge,
           ln_gamma, ln_beta, edge_index):
    src = edge_index[0].astype(_i32)
    dst = edge_index[1].astype(_i32)

    eye = jnp.eye(HEADS, dtype=_f32)
    a_src_m = (eye[:, None, :] * att_src[:, :, None]).reshape(HIDDEN, HEADS)
    a_dst_m = (eye[:, None, :] * att_dst[:, :, None]).reshape(HIDDEN, HEADS)
    a_edge_m = (eye[:, None, :] * att_edge[:, :, None]).reshape(HIDDEN, HEADS)
    a_sd = jnp.concatenate([a_src_m, a_dst_m], axis=1)

    h, a_nodes, m_node = _node_proj(node_feat, W_node, a_sd)
    ae_t, m_edge = _edge_coef(edge_attr.T, W_edge, a_edge_m)

    mb = m_node[0, :HEADS] + m_node[0, HEADS:] + m_edge[:, 0]
    mb = jnp.where(mb >= 0.0, mb, mb * 0.2)
    mb16 = jnp.broadcast_to(mb[:, None], (HEADS, L)).reshape(-1)

    asrc, adst = a_nodes[:, :HEADS], a_nodes[:, HEADS:]
    an_sp = jnp.concatenate(
        [jnp.concatenate([asrc[:, 2 * c:2 * c + 2], adst[:, 2 * c:2 * c + 2]],
                         axis=1).reshape(-1) for c in range(NC)])

    p, dparts = _pass1(src, dst, an_sp, ae_t.reshape(-1), mb16)
    dinv = _dinv(dparts.reshape(NS, NC * N_NODES * 2)).reshape(-1)
    acc1, u = _pass2(src, dst, p, h, edge_attr.reshape(-1))
    a1 = acc1.reshape(NC, SPMH_R * 2, 64)[:, :N_NODES, :]
    u2 = u.reshape(NC, SPME_R * 4, 2 * EDGE_DIM)[:, :N_NODES, :]
    return _final(a1, u2,
                  dinv.reshape(NC, N_NODES, 2), node_feat, W_edge,
                  ln_gamma.reshape(1, HIDDEN), ln_beta.reshape(1, HIDDEN))
